# Initial kernel scaffold; baseline (speedup 1.0000x reference)
import jax
import jax.numpy as jnp
from jax.experimental import pallas as pl


def kernel(x, edge_index, W1, b1, W2, b2):
    raise NotImplementedError("write your pallas kernel here")


# speedup vs baseline: 11.9225x; 11.9225x over previous
import functools

import jax
import jax.numpy as jnp
from jax import lax
from jax.experimental import pallas as pl
from jax.experimental.pallas import tpu as pltpu
from jax.experimental.pallas import tpu_sc as plsc

N_NODES = 10000
N_EDGES = 320000
DIM = 128
NC, NS = 2, 16
NW = NC * NS
PER_W = N_EDGES // NW
CH = 80
N_PAD = 10240
STRIPE = N_PAD // NS
BM = 1000

def _mesh():
    return plsc.VectorSubcoreMesh(
        core_axis_name="c", subcore_axis_name="s", num_cores=NC, num_subcores=NS
    )


def _sc_degree(dst, ones_rows, zeros_init):
    @functools.partial(
        pl.kernel,
        out_type=jax.ShapeDtypeStruct((NC, N_PAD, DIM), jnp.float32),
        mesh=_mesh(),
        scratch_types=[
            pltpu.VMEM((CH,), jnp.int32),
            pltpu.VMEM((CH, DIM), jnp.float32),
            pltpu.VMEM_SHARED((N_PAD, DIM), jnp.float32),
            pltpu.SemaphoreType.DMA,
        ],
    )
    def deg_kernel(dst_hbm, ones_hbm, zeros_hbm, out_hbm, idx_v, ones_v, acc_sh, sem):
        c = lax.axis_index("c")
        s = lax.axis_index("s")
        pltpu.sync_copy(zeros_hbm, acc_sh.at[pl.ds(s * STRIPE, STRIPE)])
        pltpu.sync_copy(ones_hbm, ones_v)
        plsc.subcore_barrier()
        base = (c * NS + s) * PER_W

        @pl.loop(0, PER_W, step=CH)
        def _(j):
            pltpu.sync_copy(dst_hbm.at[pl.ds(base + j, CH)], idx_v)
            pltpu.sync_copy(ones_v, acc_sh.at[idx_v], add=True)

        plsc.subcore_barrier()
        pltpu.sync_copy(
            acc_sh.at[pl.ds(s * STRIPE, STRIPE)],
            out_hbm.at[c, pl.ds(s * STRIPE, STRIPE)],
        )

    return deg_kernel(dst, ones_rows, zeros_init)


def _sc_aggregate(h, src, dst, zeros_init):
    @functools.partial(
        pl.kernel,
        out_type=jax.ShapeDtypeStruct((NC, N_PAD, DIM), jnp.float32),
        mesh=_mesh(),
        scratch_types=[
            pltpu.VMEM((CH,), jnp.int32),
            pltpu.VMEM((CH,), jnp.int32),
            pltpu.VMEM((CH, DIM), jnp.float32),
            pltpu.VMEM_SHARED((N_PAD, DIM), jnp.float32),
            pltpu.SemaphoreType.DMA,
        ],
    )
    def agg_kernel(h_hbm, src_hbm, dst_hbm, zeros_hbm, out_hbm,
                   src_v, dst_v, rows_v, acc_sh, sem):
        c = lax.axis_index("c")
        s = lax.axis_index("s")
        pltpu.sync_copy(zeros_hbm, acc_sh.at[pl.ds(s * STRIPE, STRIPE)])
        plsc.subcore_barrier()
        base = (c * NS + s) * PER_W

        @pl.loop(0, PER_W, step=CH)
        def _(j):
            pltpu.sync_copy(src_hbm.at[pl.ds(base + j, CH)], src_v)
            pltpu.sync_copy(dst_hbm.at[pl.ds(base + j, CH)], dst_v)
            pltpu.async_copy(h_hbm.at[src_v], rows_v, sem).wait()
            pltpu.sync_copy(rows_v, acc_sh.at[dst_v], add=True)

        plsc.subcore_barrier()
        pltpu.sync_copy(
            acc_sh.at[pl.ds(s * STRIPE, STRIPE)],
            out_hbm.at[c, pl.ds(s * STRIPE, STRIPE)],
        )

    return agg_kernel(h, src, dst, zeros_init)


def _dinv_block(deg_ref):
    d = deg_ref[0] + deg_ref[1]
    return lax.rsqrt(d[:, 0:1] + 1.0)


def _tc_in_body(deg_ref, x_ref, w_ref, out_ref):
    dinv = _dinv_block(deg_ref)
    h = jnp.dot(x_ref[...], w_ref[...], preferred_element_type=jnp.float32)
    out_ref[...] = h * dinv


def _tc_mid_body(deg_ref, p_ref, hp_ref, b_ref, w_ref, out_ref):
    dinv = _dinv_block(deg_ref)
    t = dinv * (p_ref[0] + p_ref[1] + hp_ref[...]) + b_ref[...]
    r = jnp.maximum(t, 0.0)
    h = jnp.dot(r, w_ref[...], preferred_element_type=jnp.float32)
    out_ref[...] = h * dinv


def _tc_out_body(deg_ref, p_ref, hp_ref, b_ref, out_ref):
    dinv = _dinv_block(deg_ref)
    t = dinv * (p_ref[0] + p_ref[1] + hp_ref[...]) + b_ref[...]
    m = jnp.max(t, axis=1, keepdims=True)
    lse = jnp.log(jnp.sum(jnp.exp(t - m), axis=1, keepdims=True)) + m
    out_ref[...] = t - lse


_DEG_SPEC = pl.BlockSpec((NC, BM, 16), lambda i: (0, i, 0))
_ROW_SPEC = pl.BlockSpec((BM, DIM), lambda i: (i, 0))
_PART_SPEC = pl.BlockSpec((NC, BM, DIM), lambda i: (0, i, 0))
_W_SPEC = pl.BlockSpec((DIM, DIM), lambda i: (0, 0))
_B_SPEC = pl.BlockSpec((1, DIM), lambda i: (0, 0))
_GRID = (N_NODES // BM,)
_ROW_OUT = jax.ShapeDtypeStruct((N_NODES, DIM), jnp.float32)


def _tc_in(deg, x, W1):
    return pl.pallas_call(
        _tc_in_body,
        grid=_GRID,
        in_specs=[_DEG_SPEC, _ROW_SPEC, _W_SPEC],
        out_specs=_ROW_SPEC,
        out_shape=_ROW_OUT,
    )(deg, x, W1)


def _tc_mid(deg, parts, hp, b1, W2):
    return pl.pallas_call(
        _tc_mid_body,
        grid=_GRID,
        in_specs=[_DEG_SPEC, _PART_SPEC, _ROW_SPEC, _B_SPEC, _W_SPEC],
        out_specs=_ROW_SPEC,
        out_shape=_ROW_OUT,
    )(deg, parts, hp, b1, W2)


def _tc_out(deg, parts, hp, b2):
    return pl.pallas_call(
        _tc_out_body,
        grid=_GRID,
        in_specs=[_DEG_SPEC, _PART_SPEC, _ROW_SPEC, _B_SPEC],
        out_specs=_ROW_SPEC,
        out_shape=_ROW_OUT,
    )(deg, parts, hp, b2)


def kernel(x, edge_index, W1, b1, W2, b2):
    ei = edge_index.astype(jnp.int32)
    src = ei[0]
    dst = ei[1]
    ones128 = jnp.ones((CH, DIM), jnp.float32)
    zeros128 = jnp.zeros((STRIPE, DIM), jnp.float32)

    deg = _sc_degree(dst, ones128, zeros128)[:, :N_NODES, :16]
    h1p = _tc_in(deg, x, W1)
    p1 = _sc_aggregate(h1p, src, dst, zeros128)[:, :N_NODES]
    h2p = _tc_mid(deg, p1, h1p, b1.reshape(1, DIM), W2)
    p2 = _sc_aggregate(h2p, src, dst, zeros128)[:, :N_NODES]
    return _tc_out(deg, p2, h2p, b2.reshape(1, DIM))


# speedup vs baseline: 24.7866x; 2.0790x over previous
import functools

import jax
import jax.numpy as jnp
from jax import lax
from jax.experimental import pallas as pl
from jax.experimental.pallas import tpu as pltpu
from jax.experimental.pallas import tpu_sc as plsc

N_NODES = 10000
N_EDGES = 320000
DIM = 128
NC, NS = 2, 16
NW = NC * NS
PER_W = N_EDGES // NW
CH = 80
NITER = PER_W // CH
N_PAD = 10240
STRIPE = N_PAD // NS
BM = 1000


def _mesh():
    return plsc.VectorSubcoreMesh(
        core_axis_name="c", subcore_axis_name="s", num_cores=NC, num_subcores=NS
    )


def _sc_degree(dst3, ones_rows, zeros_init):
    @functools.partial(
        pl.kernel,
        out_type=jax.ShapeDtypeStruct((NC, N_PAD, DIM), jnp.float32),
        mesh=_mesh(),
        scratch_types=[
            pltpu.VMEM((CH,), jnp.int32),
            pltpu.VMEM((CH,), jnp.int32),
            pltpu.VMEM((CH, DIM), jnp.float32),
            pltpu.VMEM_SHARED((N_PAD, DIM), jnp.float32),
            pltpu.SemaphoreType.DMA,
            pltpu.SemaphoreType.DMA,
        ],
    )
    def deg_kernel(dst_hbm, ones_hbm, zeros_hbm, out_hbm,
                   di0, di1, ones_v, acc_sh, sem0, sem1):
        c = lax.axis_index("c")
        s = lax.axis_index("s")
        base = (c * NS + s) * PER_W
        pltpu.sync_copy(zeros_hbm, acc_sh.at[pl.ds(s * STRIPE, STRIPE)])
        pltpu.sync_copy(ones_hbm, ones_v)
        plsc.subcore_barrier()

        def start(j, dbuf, sem):
            pltpu.async_copy(dst_hbm.at[pl.ds(base + j * CH, CH)], dbuf, sem)

        def drain(dbuf, sem):
            pltpu.make_async_copy(dst_hbm.at[pl.ds(0, CH)], dbuf, sem).wait()
            pltpu.sync_copy(ones_v, acc_sh.at[dbuf], add=True)

        start(0, di0, sem0)

        @pl.loop(0, NITER - 1, step=2)
        def _(j):
            start(j + 1, di1, sem1)
            drain(di0, sem0)
            start(j + 2, di0, sem0)
            drain(di1, sem1)

        drain(di0, sem0)

        plsc.subcore_barrier()
        pltpu.sync_copy(
            acc_sh.at[pl.ds(s * STRIPE, STRIPE)],
            out_hbm.at[c, pl.ds(s * STRIPE, STRIPE)],
        )

    return deg_kernel(dst3, ones_rows, zeros_init)


def _sc_aggregate(h, src2, dst3, zeros_init):
    @functools.partial(
        pl.kernel,
        out_type=jax.ShapeDtypeStruct((NC, N_PAD, DIM), jnp.float32),
        mesh=_mesh(),
        scratch_types=[
            pltpu.VMEM((PER_W,), jnp.int32),
            pltpu.VMEM((CH,), jnp.int32),
            pltpu.VMEM((CH,), jnp.int32),
            pltpu.VMEM((CH, DIM), jnp.float32),
            pltpu.VMEM((CH, DIM), jnp.float32),
            pltpu.VMEM_SHARED((N_PAD, DIM), jnp.float32),
            pltpu.SemaphoreType.DMA,
            pltpu.SemaphoreType.DMA,
            pltpu.SemaphoreType.DMA,
            pltpu.SemaphoreType.DMA,
        ],
    )
    def agg_kernel(h_hbm, src_hbm, dst_hbm, zeros_hbm, out_hbm,
                   src_v, di0, di1, buf0, buf1, acc_sh, gsem0, gsem1, isem0, isem1):
        c = lax.axis_index("c")
        s = lax.axis_index("s")
        w = c * NS + s
        base = w * PER_W
        pltpu.sync_copy(zeros_hbm, acc_sh.at[pl.ds(s * STRIPE, STRIPE)])
        pltpu.sync_copy(src_hbm.at[w], src_v)
        plsc.subcore_barrier()

        def start(j, buf, dbuf, gsem, isem):
            off = pl.multiple_of(j * CH, 8)
            pltpu.async_copy(h_hbm.at[src_v.at[pl.ds(off, CH)]], buf, gsem)
            pltpu.async_copy(dst_hbm.at[pl.ds(base + j * CH, CH)], dbuf, isem)

        def drain(buf, dbuf, gsem, isem):
            pltpu.make_async_copy(h_hbm.at[src_v.at[pl.ds(0, CH)]], buf, gsem).wait()
            pltpu.make_async_copy(dst_hbm.at[pl.ds(0, CH)], dbuf, isem).wait()
            pltpu.sync_copy(buf, acc_sh.at[dbuf], add=True)

        start(0, buf0, di0, gsem0, isem0)

        @pl.loop(0, NITER - 1, step=2)
        def _(j):
            start(j + 1, buf1, di1, gsem1, isem1)
            drain(buf0, di0, gsem0, isem0)
            start(j + 2, buf0, di0, gsem0, isem0)
            drain(buf1, di1, gsem1, isem1)

        drain(buf0, di0, gsem0, isem0)

        plsc.subcore_barrier()
        pltpu.sync_copy(
            acc_sh.at[pl.ds(s * STRIPE, STRIPE)],
            out_hbm.at[c, pl.ds(s * STRIPE, STRIPE)],
        )

    return agg_kernel(h, src2, dst3, zeros_init)


def _dinv_block(deg_ref):
    d = deg_ref[0] + deg_ref[1]
    return lax.rsqrt(d[:, 0:1] + 1.0)


def _tc_in_body(deg_ref, x_ref, w_ref, out_ref):
    dinv = _dinv_block(deg_ref)
    h = jnp.dot(x_ref[...], w_ref[...], preferred_element_type=jnp.float32)
    out_ref[...] = h * dinv


def _tc_mid_body(deg_ref, p_ref, hp_ref, b_ref, w_ref, out_ref):
    dinv = _dinv_block(deg_ref)
    t = dinv * (p_ref[0] + p_ref[1] + hp_ref[...]) + b_ref[...]
    r = jnp.maximum(t, 0.0)
    h = jnp.dot(r, w_ref[...], preferred_element_type=jnp.float32)
    out_ref[...] = h * dinv


def _tc_out_body(deg_ref, p_ref, hp_ref, b_ref, out_ref):
    dinv = _dinv_block(deg_ref)
    t = dinv * (p_ref[0] + p_ref[1] + hp_ref[...]) + b_ref[...]
    m = jnp.max(t, axis=1, keepdims=True)
    lse = jnp.log(jnp.sum(jnp.exp(t - m), axis=1, keepdims=True)) + m
    out_ref[...] = t - lse


_DEG_SPEC = pl.BlockSpec((NC, BM, 16), lambda i: (0, i, 0))
_ROW_SPEC = pl.BlockSpec((BM, DIM), lambda i: (i, 0))
_PART_SPEC = pl.BlockSpec((NC, BM, DIM), lambda i: (0, i, 0))
_W_SPEC = pl.BlockSpec((DIM, DIM), lambda i: (0, 0))
_B_SPEC = pl.BlockSpec((1, DIM), lambda i: (0, 0))
_GRID = (N_NODES // BM,)
_ROW_OUT = jax.ShapeDtypeStruct((N_NODES, DIM), jnp.float32)


def _tc_in(deg, x, W1):
    return pl.pallas_call(
        _tc_in_body,
        grid=_GRID,
        in_specs=[_DEG_SPEC, _ROW_SPEC, _W_SPEC],
        out_specs=_ROW_SPEC,
        out_shape=_ROW_OUT,
    )(deg, x, W1)


def _tc_mid(deg, parts, hp, b1, W2):
    return pl.pallas_call(
        _tc_mid_body,
        grid=_GRID,
        in_specs=[_DEG_SPEC, _PART_SPEC, _ROW_SPEC, _B_SPEC, _W_SPEC],
        out_specs=_ROW_SPEC,
        out_shape=_ROW_OUT,
    )(deg, parts, hp, b1, W2)


def _tc_out(deg, parts, hp, b2):
    return pl.pallas_call(
        _tc_out_body,
        grid=_GRID,
        in_specs=[_DEG_SPEC, _PART_SPEC, _ROW_SPEC, _B_SPEC],
        out_specs=_ROW_SPEC,
        out_shape=_ROW_OUT,
    )(deg, parts, hp, b2)


def kernel(x, edge_index, W1, b1, W2, b2):
    ei = edge_index.astype(jnp.int32)
    src2 = ei[0].reshape(NW, PER_W)
    dst1 = ei[1]
    ones128 = jnp.ones((CH, DIM), jnp.float32)
    zeros128 = jnp.zeros((STRIPE, DIM), jnp.float32)

    deg = _sc_degree(dst1, ones128, zeros128)[:, :N_NODES, :16]
    h1p = _tc_in(deg, x, W1)
    p1 = _sc_aggregate(h1p, src2, dst1, zeros128)[:, :N_NODES]
    h2p = _tc_mid(deg, p1, h1p, b1.reshape(1, DIM), W2)
    p2 = _sc_aggregate(h2p, src2, dst1, zeros128)[:, :N_NODES]
    return _tc_out(deg, p2, h2p, b2.reshape(1, DIM))


# speedup vs baseline: 29.5121x; 1.1906x over previous
import functools

import jax
import jax.numpy as jnp
from jax import lax
from jax.experimental import pallas as pl
from jax.experimental.pallas import tpu as pltpu
from jax.experimental.pallas import tpu_sc as plsc

N_NODES = 10000
N_EDGES = 320000
DIM = 128
NC, NS = 2, 16
NW = NC * NS
PER_W = N_EDGES // NW
CH = 80
NITER = PER_W // CH
N_PAD = 10240
STRIPE = N_PAD // NS
BM = 1000


def _mesh():
    return plsc.VectorSubcoreMesh(
        core_axis_name="c", subcore_axis_name="s", num_cores=NC, num_subcores=NS
    )


def _sc_degree(dst3, ones_rows, zeros_init):
    @functools.partial(
        pl.kernel,
        out_type=jax.ShapeDtypeStruct((NC, N_PAD, DIM), jnp.float32),
        mesh=_mesh(),
        scratch_types=[
            pltpu.VMEM((CH,), jnp.int32),
            pltpu.VMEM((CH,), jnp.int32),
            pltpu.VMEM((CH, DIM), jnp.float32),
            pltpu.VMEM_SHARED((N_PAD, DIM), jnp.float32),
            pltpu.SemaphoreType.DMA,
            pltpu.SemaphoreType.DMA,
        ],
    )
    def deg_kernel(dst_hbm, ones_hbm, zeros_hbm, out_hbm,
                   di0, di1, ones_v, acc_sh, sem0, sem1):
        c = lax.axis_index("c")
        s = lax.axis_index("s")
        base = (c * NS + s) * PER_W
        pltpu.sync_copy(zeros_hbm, acc_sh.at[pl.ds(s * STRIPE, STRIPE)])
        pltpu.sync_copy(ones_hbm, ones_v)
        plsc.subcore_barrier()

        def start(j, dbuf, sem):
            pltpu.async_copy(dst_hbm.at[pl.ds(base + j * CH, CH)], dbuf, sem)

        def drain(dbuf, sem):
            pltpu.make_async_copy(dst_hbm.at[pl.ds(0, CH)], dbuf, sem).wait()
            pltpu.sync_copy(ones_v, acc_sh.at[dbuf], add=True)

        start(0, di0, sem0)

        @pl.loop(0, NITER - 1, step=2)
        def _(j):
            start(j + 1, di1, sem1)
            drain(di0, sem0)
            start(j + 2, di0, sem0)
            drain(di1, sem1)

        drain(di0, sem0)

        plsc.subcore_barrier()
        pltpu.sync_copy(
            acc_sh.at[pl.ds(s * STRIPE, STRIPE)],
            out_hbm.at[c, pl.ds(s * STRIPE, STRIPE)],
        )

    return deg_kernel(dst3, ones_rows, zeros_init)


def _sc_aggregate(h, src2, dst3, zeros_init):
    @functools.partial(
        pl.kernel,
        out_type=jax.ShapeDtypeStruct((NC, N_PAD, DIM), jnp.float32),
        mesh=_mesh(),
        scratch_types=[
            pltpu.VMEM((PER_W,), jnp.int32),
            pltpu.VMEM((CH,), jnp.int32),
            pltpu.VMEM((CH,), jnp.int32),
            pltpu.VMEM((CH,), jnp.int32),
            pltpu.VMEM((CH, DIM), jnp.float32),
            pltpu.VMEM((CH, DIM), jnp.float32),
            pltpu.VMEM((CH, DIM), jnp.float32),
            pltpu.VMEM_SHARED((N_PAD, DIM), jnp.float32),
            pltpu.SemaphoreType.DMA,
            pltpu.SemaphoreType.DMA,
            pltpu.SemaphoreType.DMA,
            pltpu.SemaphoreType.DMA,
            pltpu.SemaphoreType.DMA,
            pltpu.SemaphoreType.DMA,
        ],
    )
    def agg_kernel(h_hbm, src_hbm, dst_hbm, zeros_hbm, out_hbm,
                   src_v, di0, di1, di2, buf0, buf1, buf2, acc_sh,
                   gsem0, gsem1, gsem2, isem0, isem1, isem2):
        c = lax.axis_index("c")
        s = lax.axis_index("s")
        w = c * NS + s
        base = w * PER_W
        pltpu.sync_copy(zeros_hbm, acc_sh.at[pl.ds(s * STRIPE, STRIPE)])
        pltpu.sync_copy(src_hbm.at[w], src_v)
        plsc.subcore_barrier()

        def start(j, buf, dbuf, gsem, isem):
            off = pl.multiple_of(j * CH, 8)
            pltpu.async_copy(h_hbm.at[src_v.at[pl.ds(off, CH)]], buf, gsem)
            pltpu.async_copy(dst_hbm.at[pl.ds(base + j * CH, CH)], dbuf, isem)

        def drain(buf, dbuf, gsem, isem):
            pltpu.make_async_copy(h_hbm.at[src_v.at[pl.ds(0, CH)]], buf, gsem).wait()
            pltpu.make_async_copy(dst_hbm.at[pl.ds(0, CH)], dbuf, isem).wait()
            pltpu.sync_copy(buf, acc_sh.at[dbuf], add=True)

        start(0, buf0, di0, gsem0, isem0)
        start(1, buf1, di1, gsem1, isem1)
        start(2, buf2, di2, gsem2, isem2)

        @pl.loop(0, NITER - 4, step=3)
        def _(j):
            drain(buf0, di0, gsem0, isem0)
            start(j + 3, buf0, di0, gsem0, isem0)
            drain(buf1, di1, gsem1, isem1)
            start(j + 4, buf1, di1, gsem1, isem1)
            drain(buf2, di2, gsem2, isem2)

            @pl.when(j + 5 < NITER)
            def _():
                start(j + 5, buf2, di2, gsem2, isem2)

        drain(buf0, di0, gsem0, isem0)
        drain(buf1, di1, gsem1, isem1)

        plsc.subcore_barrier()
        pltpu.sync_copy(
            acc_sh.at[pl.ds(s * STRIPE, STRIPE)],
            out_hbm.at[c, pl.ds(s * STRIPE, STRIPE)],
        )

    return agg_kernel(h, src2, dst3, zeros_init)


def _dinv_block(deg_ref):
    d = deg_ref[0] + deg_ref[1]
    return lax.rsqrt(d[:, 0:1] + 1.0)


def _tc_in_body(deg_ref, x_ref, w_ref, out_ref):
    dinv = _dinv_block(deg_ref)
    h = jnp.dot(x_ref[...], w_ref[...], preferred_element_type=jnp.float32)
    out_ref[...] = h * dinv


def _tc_mid_body(deg_ref, p_ref, hp_ref, b_ref, w_ref, out_ref):
    dinv = _dinv_block(deg_ref)
    t = dinv * (p_ref[0] + p_ref[1] + hp_ref[...]) + b_ref[...]
    r = jnp.maximum(t, 0.0)
    h = jnp.dot(r, w_ref[...], preferred_element_type=jnp.float32)
    out_ref[...] = h * dinv


def _tc_out_body(deg_ref, p_ref, hp_ref, b_ref, out_ref):
    dinv = _dinv_block(deg_ref)
    t = dinv * (p_ref[0] + p_ref[1] + hp_ref[...]) + b_ref[...]
    m = jnp.max(t, axis=1, keepdims=True)
    lse = jnp.log(jnp.sum(jnp.exp(t - m), axis=1, keepdims=True)) + m
    out_ref[...] = t - lse


_DEG_SPEC = pl.BlockSpec((NC, BM, 16), lambda i: (0, i, 0))
_ROW_SPEC = pl.BlockSpec((BM, DIM), lambda i: (i, 0))
_PART_SPEC = pl.BlockSpec((NC, BM, DIM), lambda i: (0, i, 0))
_W_SPEC = pl.BlockSpec((DIM, DIM), lambda i: (0, 0))
_B_SPEC = pl.BlockSpec((1, DIM), lambda i: (0, 0))
_GRID = (N_NODES // BM,)
_ROW_OUT = jax.ShapeDtypeStruct((N_NODES, DIM), jnp.float32)


def _tc_in(deg, x, W1):
    return pl.pallas_call(
        _tc_in_body,
        grid=_GRID,
        in_specs=[_DEG_SPEC, _ROW_SPEC, _W_SPEC],
        out_specs=_ROW_SPEC,
        out_shape=_ROW_OUT,
    )(deg, x, W1)


def _tc_mid(deg, parts, hp, b1, W2):
    return pl.pallas_call(
        _tc_mid_body,
        grid=_GRID,
        in_specs=[_DEG_SPEC, _PART_SPEC, _ROW_SPEC, _B_SPEC, _W_SPEC],
        out_specs=_ROW_SPEC,
        out_shape=_ROW_OUT,
    )(deg, parts, hp, b1, W2)


def _tc_out(deg, parts, hp, b2):
    return pl.pallas_call(
        _tc_out_body,
        grid=_GRID,
        in_specs=[_DEG_SPEC, _PART_SPEC, _ROW_SPEC, _B_SPEC],
        out_specs=_ROW_SPEC,
        out_shape=_ROW_OUT,
    )(deg, parts, hp, b2)


def kernel(x, edge_index, W1, b1, W2, b2):
    ei = edge_index.astype(jnp.int32)
    src2 = ei[0].reshape(NW, PER_W)
    dst1 = ei[1]
    ones128 = jnp.ones((CH, DIM), jnp.float32)
    zeros128 = jnp.zeros((STRIPE, DIM), jnp.float32)

    deg = _sc_degree(dst1, ones128, zeros128)[:, :, :16]
    h1p = _tc_in(deg, x, W1)
    p1 = _sc_aggregate(h1p, src2, dst1, zeros128)
    h2p = _tc_mid(deg, p1, h1p, b1.reshape(1, DIM), W2)
    p2 = _sc_aggregate(h2p, src2, dst1, zeros128)
    return _tc_out(deg, p2, h2p, b2.reshape(1, DIM))


# speedup vs baseline: 31.3013x; 1.0606x over previous
import functools

import jax
import jax.numpy as jnp
from jax import lax
from jax.experimental import pallas as pl
from jax.experimental.pallas import tpu as pltpu
from jax.experimental.pallas import tpu_sc as plsc

N_NODES = 10000
N_EDGES = 320000
DIM = 128
NC, NS = 2, 16
NW = NC * NS
PER_W = N_EDGES // NW
CH = 80
NITER = PER_W // CH
N_PAD = 10240
STRIPE = N_PAD // NS
BM = 1000


def _mesh():
    return plsc.VectorSubcoreMesh(
        core_axis_name="c", subcore_axis_name="s", num_cores=NC, num_subcores=NS
    )


def _sc_degree(dst3, ones_rows, zeros_init):
    @functools.partial(
        pl.kernel,
        out_type=jax.ShapeDtypeStruct((NC, N_PAD, 16), jnp.float32),
        mesh=_mesh(),
        scratch_types=[
            pltpu.VMEM((CH,), jnp.int32),
            pltpu.VMEM((CH,), jnp.int32),
            pltpu.VMEM((CH, 16), jnp.float32),
            pltpu.VMEM_SHARED((N_PAD, 16), jnp.float32),
            pltpu.SemaphoreType.DMA,
            pltpu.SemaphoreType.DMA,
        ],
    )
    def deg_kernel(dst_hbm, ones_hbm, zeros_hbm, out_hbm,
                   di0, di1, ones_v, acc_sh, sem0, sem1):
        c = lax.axis_index("c")
        s = lax.axis_index("s")
        base = (c * NS + s) * PER_W
        pltpu.sync_copy(zeros_hbm, acc_sh.at[pl.ds(s * STRIPE, STRIPE)])
        pltpu.sync_copy(ones_hbm, ones_v)
        plsc.subcore_barrier()

        def start(j, dbuf, sem):
            pltpu.async_copy(dst_hbm.at[pl.ds(base + j * CH, CH)], dbuf, sem)

        def drain(dbuf, sem):
            pltpu.make_async_copy(dst_hbm.at[pl.ds(0, CH)], dbuf, sem).wait()
            pltpu.sync_copy(ones_v, acc_sh.at[dbuf], add=True)

        start(0, di0, sem0)

        @pl.loop(0, NITER - 1, step=2)
        def _(j):
            start(j + 1, di1, sem1)
            drain(di0, sem0)
            start(j + 2, di0, sem0)
            drain(di1, sem1)

        drain(di0, sem0)

        plsc.subcore_barrier()
        pltpu.sync_copy(
            acc_sh.at[pl.ds(s * STRIPE, STRIPE)],
            out_hbm.at[c, pl.ds(s * STRIPE, STRIPE)],
        )

    return deg_kernel(dst3, ones_rows, zeros_init)


def _sc_aggregate(h, src2, dst3, zeros_init):
    @functools.partial(
        pl.kernel,
        out_type=jax.ShapeDtypeStruct((NC, N_PAD, DIM), jnp.float32),
        mesh=_mesh(),
        scratch_types=[
            pltpu.VMEM((PER_W,), jnp.int32),
            pltpu.VMEM((CH,), jnp.int32),
            pltpu.VMEM((CH,), jnp.int32),
            pltpu.VMEM((CH,), jnp.int32),
            pltpu.VMEM((CH, DIM), jnp.float32),
            pltpu.VMEM((CH, DIM), jnp.float32),
            pltpu.VMEM((CH, DIM), jnp.float32),
            pltpu.VMEM_SHARED((N_PAD, DIM), jnp.float32),
            pltpu.SemaphoreType.DMA,
            pltpu.SemaphoreType.DMA,
            pltpu.SemaphoreType.DMA,
            pltpu.SemaphoreType.DMA,
            pltpu.SemaphoreType.DMA,
            pltpu.SemaphoreType.DMA,
        ],
    )
    def agg_kernel(h_hbm, src_hbm, dst_hbm, zeros_hbm, out_hbm,
                   src_v, di0, di1, di2, buf0, buf1, buf2, acc_sh,
                   gsem0, gsem1, gsem2, isem0, isem1, isem2):
        c = lax.axis_index("c")
        s = lax.axis_index("s")
        w = c * NS + s
        base = w * PER_W
        pltpu.sync_copy(zeros_hbm, acc_sh.at[pl.ds(s * STRIPE, STRIPE)])
        pltpu.sync_copy(src_hbm.at[w], src_v)
        plsc.subcore_barrier()

        def start(j, buf, dbuf, gsem, isem):
            off = pl.multiple_of(j * CH, 8)
            pltpu.async_copy(h_hbm.at[src_v.at[pl.ds(off, CH)]], buf, gsem)
            pltpu.async_copy(dst_hbm.at[pl.ds(base + j * CH, CH)], dbuf, isem)

        def drain(buf, dbuf, gsem, isem):
            pltpu.make_async_copy(h_hbm.at[src_v.at[pl.ds(0, CH)]], buf, gsem).wait()
            pltpu.make_async_copy(dst_hbm.at[pl.ds(0, CH)], dbuf, isem).wait()
            pltpu.sync_copy(buf, acc_sh.at[dbuf], add=True)

        start(0, buf0, di0, gsem0, isem0)
        start(1, buf1, di1, gsem1, isem1)
        start(2, buf2, di2, gsem2, isem2)

        @pl.loop(0, NITER - 4, step=3)
        def _(j):
            drain(buf0, di0, gsem0, isem0)
            start(j + 3, buf0, di0, gsem0, isem0)
            drain(buf1, di1, gsem1, isem1)
            start(j + 4, buf1, di1, gsem1, isem1)
            drain(buf2, di2, gsem2, isem2)

            @pl.when(j + 5 < NITER)
            def _():
                start(j + 5, buf2, di2, gsem2, isem2)

        drain(buf0, di0, gsem0, isem0)
        drain(buf1, di1, gsem1, isem1)

        plsc.subcore_barrier()
        pltpu.sync_copy(
            acc_sh.at[pl.ds(s * STRIPE, STRIPE)],
            out_hbm.at[c, pl.ds(s * STRIPE, STRIPE)],
        )

    return agg_kernel(h, src2, dst3, zeros_init)


def _dinv_block(deg_ref):
    d = deg_ref[0] + deg_ref[1]
    return lax.rsqrt(d[:, 0:1] + 1.0)


def _tc_in_body(deg_ref, x_ref, w_ref, out_ref):
    dinv = _dinv_block(deg_ref)
    h = jnp.dot(x_ref[...], w_ref[...], preferred_element_type=jnp.float32)
    out_ref[...] = h * dinv


def _tc_mid_body(deg_ref, p_ref, hp_ref, b_ref, w_ref, out_ref):
    dinv = _dinv_block(deg_ref)
    t = dinv * (p_ref[0] + p_ref[1] + hp_ref[...]) + b_ref[...]
    r = jnp.maximum(t, 0.0)
    h = jnp.dot(r, w_ref[...], preferred_element_type=jnp.float32)
    out_ref[...] = h * dinv


def _tc_out_body(deg_ref, p_ref, hp_ref, b_ref, out_ref):
    dinv = _dinv_block(deg_ref)
    t = dinv * (p_ref[0] + p_ref[1] + hp_ref[...]) + b_ref[...]
    m = jnp.max(t, axis=1, keepdims=True)
    lse = jnp.log(jnp.sum(jnp.exp(t - m), axis=1, keepdims=True)) + m
    out_ref[...] = t - lse


_DEG_SPEC = pl.BlockSpec((NC, BM, 16), lambda i: (0, i, 0))
_ROW_SPEC = pl.BlockSpec((BM, DIM), lambda i: (i, 0))
_PART_SPEC = pl.BlockSpec((NC, BM, DIM), lambda i: (0, i, 0))
_W_SPEC = pl.BlockSpec((DIM, DIM), lambda i: (0, 0))
_B_SPEC = pl.BlockSpec((1, DIM), lambda i: (0, 0))
_GRID = (N_NODES // BM,)
_ROW_OUT = jax.ShapeDtypeStruct((N_NODES, DIM), jnp.float32)


def _tc_in(deg, x, W1):
    return pl.pallas_call(
        _tc_in_body,
        grid=_GRID,
        in_specs=[_DEG_SPEC, _ROW_SPEC, _W_SPEC],
        out_specs=_ROW_SPEC,
        out_shape=_ROW_OUT,
    )(deg, x, W1)


def _tc_mid(deg, parts, hp, b1, W2):
    return pl.pallas_call(
        _tc_mid_body,
        grid=_GRID,
        in_specs=[_DEG_SPEC, _PART_SPEC, _ROW_SPEC, _B_SPEC, _W_SPEC],
        out_specs=_ROW_SPEC,
        out_shape=_ROW_OUT,
    )(deg, parts, hp, b1, W2)


def _tc_out(deg, parts, hp, b2):
    return pl.pallas_call(
        _tc_out_body,
        grid=_GRID,
        in_specs=[_DEG_SPEC, _PART_SPEC, _ROW_SPEC, _B_SPEC],
        out_specs=_ROW_SPEC,
        out_shape=_ROW_OUT,
    )(deg, parts, hp, b2)


def kernel(x, edge_index, W1, b1, W2, b2):
    ei = edge_index.astype(jnp.int32)
    src2 = ei[0].reshape(NW, PER_W)
    dst1 = ei[1]
    ones16 = jnp.ones((CH, 16), jnp.float32)
    zeros16 = jnp.zeros((STRIPE, 16), jnp.float32)
    zeros128 = jnp.zeros((STRIPE, DIM), jnp.float32)

    deg = _sc_degree(dst1, ones16, zeros16)
    h1p = _tc_in(deg, x, W1)
    p1 = _sc_aggregate(h1p, src2, dst1, zeros128)
    h2p = _tc_mid(deg, p1, h1p, b1.reshape(1, DIM), W2)
    p2 = _sc_aggregate(h2p, src2, dst1, zeros128)
    return _tc_out(deg, p2, h2p, b2.reshape(1, DIM))
